# Initial kernel scaffold; baseline (speedup 1.0000x reference)
import jax
import jax.numpy as jnp
from jax.experimental import pallas as pl


def kernel(x, edge_index, graph_ids, fc_W, fc_b, W0, b0, W1, b1, W2, b2, eps0, eps1, eps2, last_W, last_b):
    raise NotImplementedError("write your pallas kernel here")


# speedup vs baseline: 2.6850x; 2.6850x over previous
import functools

import jax
import jax.numpy as jnp
from jax import lax
from jax.experimental import pallas as pl
from jax.experimental.pallas import tpu as pltpu
from jax.experimental.pallas import tpu_sc as plsc

_N = 10000
_E = 160000
_G = 8
_H = 512
_C = 128
_NCH = _H // _C
_NP = 10112
_RT = _NP // 16
_EB = 128
_TPB = 79
_EP = 16 * _TPB * _EB
_BM = 632

@functools.lru_cache(maxsize=None)
def _sc_mesh():
    return plsc.VectorSubcoreMesh(
        core_axis_name="c", subcore_axis_name="s",
        num_cores=2, num_subcores=16)


def _deg_body(dst_hbm, out_hbm, idx_v, ones_v, zb_v, db_v, accum):
    core = lax.axis_index("c")
    sub = lax.axis_index("s")

    def _fill(k, _):
        zb_v[pl.ds(k * 16, 16)] = jnp.zeros((16,), jnp.float32)
        return 0
    lax.fori_loop(0, 40, _fill, 0)
    for k in range(_EB // 16):
        ones_v[pl.ds(k * 16, 16)] = jnp.ones((16,), jnp.float32)

    pltpu.sync_copy(zb_v.at[pl.ds(0, _RT)], accum.at[pl.ds(sub * _RT, _RT)])
    plsc.subcore_barrier()

    tbase = sub * (_TPB * _EB)

    def _eb(b, _):
        off = tbase + b * _EB
        pltpu.sync_copy(dst_hbm.at[pl.ds(off, _EB)], idx_v)
        pltpu.sync_copy(ones_v, accum.at[idx_v], add=True)
        return 0
    lax.fori_loop(0, _TPB, _eb, 0)
    plsc.subcore_barrier()

    @pl.when(core == 0)
    def _():
        pltpu.sync_copy(accum.at[pl.ds(sub * _RT, _RT)], db_v.at[pl.ds(0, _RT)])

        def _rec(k, _):
            v = db_v[pl.ds(k * 16, 16)]
            db_v[pl.ds(k * 16, 16)] = 1.0 / jnp.maximum(v, 1.0)
            return 0
        lax.fori_loop(0, 40, _rec, 0)
        pltpu.sync_copy(db_v.at[pl.ds(0, _RT)], out_hbm.at[pl.ds(sub * _RT, _RT)])


def _deg_call(dstp):
    return pl.kernel(
        _deg_body,
        out_type=jax.ShapeDtypeStruct((_NP,), jnp.float32),
        mesh=_sc_mesh(),
        scratch_types=[
            pltpu.VMEM((_EB,), jnp.int32),
            pltpu.VMEM((_EB,), jnp.float32),
            pltpu.VMEM((640,), jnp.float32),
            pltpu.VMEM((640,), jnp.float32),
            pltpu.VMEM_SHARED((_NP,), jnp.float32),
        ],
    )(dstp)


def _agg_body(h_hbm, src_hbm, dst_hbm, zero_hbm, out_hbm,
              sidx, gidx, didx, rows, accum, sem):
    core = lax.axis_index("c")
    sub = lax.axis_index("s")
    tbase = sub * (_TPB * _EB)

    for j in range(_NCH // 2):
        rowbase = (core * (_NCH // 2) + j) * _NP
        pltpu.sync_copy(zero_hbm, accum.at[pl.ds(sub * _RT, _RT)])
        plsc.subcore_barrier()

        def _eb(b, _):
            off = tbase + b * _EB
            pltpu.sync_copy(src_hbm.at[pl.ds(off, _EB)], sidx)
            for k in range(_EB // 16):
                gidx[pl.ds(k * 16, 16)] = sidx[pl.ds(k * 16, 16)] + rowbase
            pltpu.async_copy(h_hbm.at[gidx], rows, sem).wait()
            pltpu.sync_copy(dst_hbm.at[pl.ds(off, _EB)], didx)
            pltpu.sync_copy(rows, accum.at[didx], add=True)
            return 0
        lax.fori_loop(0, _TPB, _eb, 0)
        plsc.subcore_barrier()

        pltpu.sync_copy(accum.at[pl.ds(sub * _RT, _RT)],
                        out_hbm.at[pl.ds(rowbase + sub * _RT, _RT)])
        plsc.subcore_barrier()


def _agg_call(hflat, srcp, dstp, zero_blk):
    return pl.kernel(
        _agg_body,
        out_type=jax.ShapeDtypeStruct((_NCH * _NP, _C), jnp.float32),
        mesh=_sc_mesh(),
        scratch_types=[
            pltpu.VMEM((_EB,), jnp.int32),
            pltpu.VMEM((_EB,), jnp.int32),
            pltpu.VMEM((_EB,), jnp.int32),
            pltpu.VMEM((_EB, _C), jnp.float32),
            pltpu.VMEM_SHARED((_NP, _C), jnp.float32),
            pltpu.SemaphoreType.DMA,
        ],
    )(hflat, srcp, dstp, zero_blk)


def _fc_body(x_ref, w_ref, b_ref, o_ref):
    y = jnp.dot(x_ref[...], w_ref[...], preferred_element_type=jnp.float32)
    y = jnp.maximum(y + b_ref[...], 0.0)
    for c in range(_NCH):
        o_ref[c] = y[:, c * _C:(c + 1) * _C]


def _fc(x, w, b):
    k = x.shape[1]
    return pl.pallas_call(
        _fc_body,
        grid=(_NP // _BM,),
        in_specs=[
            pl.BlockSpec((_BM, k), lambda i: (i, 0)),
            pl.BlockSpec((k, _H), lambda i: (0, 0)),
            pl.BlockSpec((1, _H), lambda i: (0, 0)),
        ],
        out_specs=pl.BlockSpec((_NCH, _BM, _C), lambda i: (0, i, 0)),
        out_shape=jax.ShapeDtypeStruct((_NCH, _NP, _C), jnp.float32),
    )(x, w, b)


def _gin_body(s_ref, h_ref, a_ref, d_ref, w_ref, b_ref, o_ref):
    s = s_ref[0, 0]
    d = d_ref[...]
    w = w_ref[...]
    acc = jnp.zeros((_BM, _H), jnp.float32)
    for c in range(_NCH):
        rst = s * h_ref[c] + d * a_ref[c]
        acc = acc + jnp.dot(rst, w[c * _C:(c + 1) * _C, :],
                            preferred_element_type=jnp.float32)
    y = jnp.maximum(acc + b_ref[...], 0.0)
    for c in range(_NCH):
        o_ref[c] = y[:, c * _C:(c + 1) * _C]


def _gin(scale, h, agg, dinv, w, b):
    return pl.pallas_call(
        _gin_body,
        grid=(_NP // _BM,),
        in_specs=[
            pl.BlockSpec(memory_space=pltpu.SMEM),
            pl.BlockSpec((_NCH, _BM, _C), lambda i: (0, i, 0)),
            pl.BlockSpec((_NCH, _BM, _C), lambda i: (0, i, 0)),
            pl.BlockSpec((_BM, 1), lambda i: (i, 0)),
            pl.BlockSpec((_H, _H), lambda i: (0, 0)),
            pl.BlockSpec((1, _H), lambda i: (0, 0)),
        ],
        out_specs=pl.BlockSpec((_NCH, _BM, _C), lambda i: (0, i, 0)),
        out_shape=jax.ShapeDtypeStruct((_NCH, _NP, _C), jnp.float32),
    )(scale, h, agg, dinv, w, b)


def _final_body(h0, h1, h2, h3, g_ref, w_ref, b_ref, o_ref, acc_ref):
    i = pl.program_id(0)

    @pl.when(i == 0)
    def _():
        acc_ref[...] = jnp.zeros_like(acc_ref)

    hs = jnp.concatenate(
        [h0[c] + h1[c] + h2[c] + h3[c] for c in range(_NCH)], axis=1)
    oh = (g_ref[...] == lax.broadcasted_iota(jnp.int32, (_BM, _G), 1)
          ).astype(jnp.float32)
    acc_ref[...] += lax.dot_general(
        oh, hs, (((0,), (0,)), ((), ())), preferred_element_type=jnp.float32)

    @pl.when(i == pl.num_programs(0) - 1)
    def _():
        o_ref[...] = jnp.dot(acc_ref[...], w_ref[...],
                             preferred_element_type=jnp.float32) + 4.0 * b_ref[...]


def _final(h0, h1, h2, h3, gid, w, b):
    hspec = pl.BlockSpec((_NCH, _BM, _C), lambda i: (0, i, 0))
    return pl.pallas_call(
        _final_body,
        grid=(_NP // _BM,),
        in_specs=[
            hspec, hspec, hspec, hspec,
            pl.BlockSpec((_BM, 1), lambda i: (i, 0)),
            pl.BlockSpec((_H, 128), lambda i: (0, 0)),
            pl.BlockSpec((1, 128), lambda i: (0, 0)),
        ],
        out_specs=pl.BlockSpec((_G, 128), lambda i: (0, 0)),
        out_shape=jax.ShapeDtypeStruct((_G, 128), jnp.float32),
        scratch_shapes=[pltpu.VMEM((_G, _H), jnp.float32)],
    )(h0, h1, h2, h3, gid, w, b)


def kernel(x, edge_index, graph_ids, fc_W, fc_b, W0, b0, W1, b1, W2, b2,
           eps0, eps1, eps2, last_W, last_b):
    src = edge_index[0]
    dst = edge_index[1]
    srcp = jnp.concatenate([src, jnp.zeros((_EP - _E,), jnp.int32)])
    dstp = jnp.concatenate([dst, jnp.full((_EP - _E,), _N, jnp.int32)])
    xp = jnp.pad(x, ((0, _NP - _N), (0, 0)))
    gidp = jnp.pad(graph_ids, (0, _NP - _N),
                   constant_values=_G).reshape(_NP, 1)
    zero_blk = jnp.zeros((_RT, _C), jnp.float32)

    dinv = _deg_call(dstp).reshape(_NP, 1)
    h = _fc(xp, fc_W, fc_b.reshape(1, _H))

    hs = [h]
    for w, b, eps in ((W0, b0, eps0), (W1, b1, eps1), (W2, b2, eps2)):
        agg = _agg_call(h.reshape(_NCH * _NP, _C), srcp, dstp, zero_blk)
        h = _gin((1.0 + eps).reshape(1, 1), h, agg.reshape(_NCH, _NP, _C),
                 dinv, w, b.reshape(1, _H))
        hs.append(h)

    return _final(hs[0], hs[1], hs[2], hs[3], gidp, last_W,
                  last_b.reshape(1, 128))
